# baseline (speedup 1.0000x reference)
import jax
import jax.numpy as jnp
from jax import lax
from jax.experimental import pallas as pl
from jax.experimental.pallas import tpu as pltpu
from jax.experimental.pallas import tpu_sc as plsc

N = 10000
E = 160000
T = 12
C = 128
TN = T * N
EPS = 1e-5

NC, NS = 2, 16
CHUNK = 64
EPT = E // NS
NCHUNK = (EPT + CHUNK - 1) // CHUNK
EPT_PAD = NCHUNK * CHUNK
NPAD = 10112
RPT = NPAD // NS
DUMMY = N + 8
TPC = T // NC

EPTD = E // (NC * NS)
NCHUNKD = (EPTD + CHUNK - 1) // CHUNK
EPTD_PAD = NCHUNKD * CHUNK


def _deg_body(dstp, ones_in, z16, deg_out, dst_buf, ones_buf, deg_acc):
    cid = lax.axis_index("c")
    sid = lax.axis_index("s")
    pltpu.sync_copy(dstp.at[cid, sid], dst_buf)
    pltpu.sync_copy(ones_in, ones_buf)
    r0 = sid * RPT
    pltpu.sync_copy(z16.at[pl.ds(r0, RPT)], deg_acc.at[pl.ds(r0, RPT)])
    plsc.subcore_barrier()

    def chunk(j, carry):
        pltpu.sync_copy(ones_buf, deg_acc.at[dst_buf.at[j]], add=True)
        return carry
    lax.fori_loop(0, NCHUNKD, chunk, 0)
    plsc.subcore_barrier()
    pltpu.sync_copy(deg_acc.at[pl.ds(r0, RPT)],
                    deg_out.at[cid, pl.ds(r0, RPT)])


_SC_KERNELS = {}


def _sc_kernels():
    if "deg" not in _SC_KERNELS:
        mesh = plsc.VectorSubcoreMesh(
            core_axis_name="c", subcore_axis_name="s",
            num_cores=NC, num_subcores=NS)
        _SC_KERNELS["deg"] = pl.kernel(
            _deg_body,
            out_type=jax.ShapeDtypeStruct((NC, NPAD, C), jnp.float32),
            mesh=mesh,
            scratch_types=[
                pltpu.VMEM((NCHUNKD, CHUNK), jnp.int32),
                pltpu.VMEM((CHUNK, C), jnp.float32),
                pltpu.VMEM_SHARED((NPAD, C), jnp.float32),
            ],
        )
        _SC_KERNELS["scatter"] = pl.kernel(
            _scatter_body,
            out_type=jax.ShapeDtypeStruct((T, NPAD, C), jnp.float32),
            mesh=mesh,
            scratch_types=[
                pltpu.VMEM((NCHUNK * CHUNK,), jnp.int32),
                pltpu.VMEM((4, CHUNK), jnp.int32),
                pltpu.VMEM((4, CHUNK), jnp.int32),
                pltpu.VMEM((4, CHUNK, C), jnp.float32),
                pltpu.VMEM_SHARED((NPAD, C), jnp.float32),
            ] + [pltpu.SemaphoreType.DMA] * 8,
        )
    return _SC_KERNELS


def _scatter_body(hp, pkp, zbf, s_out, pk_buf, sidx, didx, gbuf, acc,
                  g0, g1, g2, g3, s0, s1, s2, s3):
    cid = lax.axis_index("c")
    sid = lax.axis_index("s")
    r0 = sid * RPT
    pltpu.sync_copy(pkp.at[sid], pk_buf)
    mask = jnp.int32((1 << 14) - 1)
    gsems = (g0, g1, g2, g3)
    ssems = (s0, s1, s2, s3)

    def fire_gather(c, slot, toff):
        base = c * CHUNK
        for kk in range(CHUNK // 16):
            sl = pl.ds(kk * 16, 16)
            v = pk_buf[pl.ds(base + kk * 16, 16)]
            sidx[slot, sl] = (v & mask) + toff
            didx[slot, sl] = lax.shift_right_logical(v, 14)
        pltpu.async_copy(hp.at[sidx.at[slot]], gbuf.at[slot], gsems[slot])

    def wait_gather(slot):
        pltpu.make_async_copy(hp.at[sidx.at[slot]], gbuf.at[slot],
                              gsems[slot]).wait()

    def fire_scatter(slot):
        pltpu.async_copy(gbuf.at[slot], acc.at[didx.at[slot]], ssems[slot],
                         add=True)

    def wait_scatter(slot):
        pltpu.make_async_copy(gbuf.at[slot], acc.at[didx.at[slot]],
                              ssems[slot]).wait()

    for tl in range(TPC):
        t = cid * TPC + tl
        toff = t * jnp.int32(N)
        pltpu.sync_copy(zbf.at[pl.ds(r0, RPT)], acc.at[pl.ds(r0, RPT)])
        plsc.subcore_barrier()

        fire_gather(0, 0, toff)
        fire_gather(1, 1, toff)
        fire_gather(2, 2, toff)

        def quad(i, carry):
            c = 4 * i
            for off in range(4):
                cc = c + off
                slot = off
                nslot = (off + 3) % 4

                @pl.when(cc < NCHUNK)
                def _():
                    wait_gather(slot)
                    fire_scatter(slot)

                @pl.when((cc + 3 < NCHUNK) & (cc >= 1))
                def _():
                    wait_scatter(nslot)
                    fire_gather(cc + 3, nslot, toff)

                @pl.when((cc + 3 < NCHUNK) & (cc < 1))
                def _():
                    fire_gather(cc + 3, nslot, toff)
            return carry
        lax.fori_loop(0, (NCHUNK + 3) // 4, quad, 0)
        for slot in range(4):
            wait_scatter(slot)
        plsc.subcore_barrier()
        pltpu.sync_copy(acc.at[pl.ds(r0, RPT)], s_out.at[t, pl.ds(r0, RPT)])


RB = 2000


def _lin_body(x_ref, w_ref, deg_ref, o_ref):
    dinv = lax.rsqrt(deg_ref[0, :, 0:1] + deg_ref[1, :, 0:1] + 1.0)
    h = lax.dot_general(x_ref[...], w_ref[...], (((1,), (0,)), ((), ())),
                        precision=lax.Precision.DEFAULT,
                        preferred_element_type=jnp.float32)
    o_ref[...] = h * dinv


_lin = pl.pallas_call(
    _lin_body,
    grid=(N // RB, T),
    in_specs=[
        pl.BlockSpec((RB, C), lambda j, t: (t * (N // RB) + j, 0)),
        pl.BlockSpec((C, C), lambda j, t: (0, 0)),
        pl.BlockSpec((2, RB, C), lambda j, t: (0, j, 0)),
    ],
    out_specs=pl.BlockSpec((RB, C), lambda j, t: (t * (N // RB) + j, 0)),
    out_shape=jax.ShapeDtypeStruct((TN, C), jnp.float32),
)


def _bn_s_body(s_ref, hp_ref, deg_ref, b_ref, gam_ref, bet_ref, o_ref):
    dinv = lax.rsqrt(deg_ref[0, 0:N, 0:1] + deg_ref[1, 0:N, 0:1] + 1.0)
    sf = s_ref[0, 0:N, :].astype(jnp.float32)
    hf = hp_ref[...].astype(jnp.float32)
    g = dinv * (sf + hf) + b_ref[...][None, :]
    mean = jnp.mean(g, axis=0, keepdims=True)
    var = jnp.mean(jnp.square(g - mean), axis=0, keepdims=True)
    hs = (g - mean) * lax.rsqrt(var + EPS) * gam_ref[...][None, :] \
        + bet_ref[...][None, :]
    o_ref[...] = jnp.maximum(hs, 0.0)


_bn_s = pl.pallas_call(
    _bn_s_body,
    grid=(T,),
    in_specs=[
        pl.BlockSpec((1, NPAD, C), lambda t: (t, 0, 0)),
        pl.BlockSpec((N, C), lambda t: (t, 0)),
        pl.BlockSpec((2, NPAD, C), lambda t: (0, 0, 0)),
        pl.BlockSpec((C,), lambda t: (0,)),
        pl.BlockSpec((C,), lambda t: (0,)),
        pl.BlockSpec((C,), lambda t: (0,)),
    ],
    out_specs=pl.BlockSpec((N, C), lambda t: (t, 0)),
    out_shape=jax.ShapeDtypeStruct((TN, C), jnp.float32),
)


NB = 5
RBT = N // NB


def _tconv_body(hm_ref, h0_ref, hp1_ref, wk_ref, cb_ref, y_ref, st_ref):
    t = pl.program_id(0)
    acc = lax.dot_general(h0_ref[...], wk_ref[1], (((1,), (0,)), ((), ())),
                          precision=lax.Precision.DEFAULT,
                          preferred_element_type=jnp.float32)
    accm = lax.dot_general(hm_ref[...], wk_ref[0], (((1,), (0,)), ((), ())),
                           precision=lax.Precision.DEFAULT,
                           preferred_element_type=jnp.float32)
    accp = lax.dot_general(hp1_ref[...], wk_ref[2], (((1,), (0,)), ((), ())),
                           precision=lax.Precision.DEFAULT,
                           preferred_element_type=jnp.float32)
    zf = jnp.float32(0.0)
    acc = acc + cb_ref[...][None, :] \
        + jnp.where(t > 0, jnp.float32(1.0), zf) * accm \
        + jnp.where(t < T - 1, jnp.float32(1.0), zf) * accp
    y_ref[...] = acc
    s1 = jnp.sum(acc, axis=0)
    s2 = jnp.sum(acc * acc, axis=0)
    st_ref[0, 0] = jnp.concatenate(
        [s1[None, :], s2[None, :], jnp.zeros((6, C), jnp.float32)], axis=0)


_tconv = pl.pallas_call(
    _tconv_body,
    grid=(T, NB),
    in_specs=[
        pl.BlockSpec((RBT, C), lambda t, b: (jnp.maximum(t - 1, 0) * NB + b, 0)),
        pl.BlockSpec((RBT, C), lambda t, b: (t * NB + b, 0)),
        pl.BlockSpec((RBT, C), lambda t, b: (jnp.minimum(t + 1, T - 1) * NB + b, 0)),
        pl.BlockSpec((3, C, C), lambda t, b: (0, 0, 0)),
        pl.BlockSpec((C,), lambda t, b: (0,)),
    ],
    out_specs=[
        pl.BlockSpec((RBT, C), lambda t, b: (t * NB + b, 0)),
        pl.BlockSpec((1, 1, 8, C), lambda t, b: (t, b, 0, 0)),
    ],
    out_shape=[
        jax.ShapeDtypeStruct((TN, C), jnp.float32),
        jax.ShapeDtypeStruct((T, NB, 8, C), jnp.float32),
    ],
)


def _bn_t_body(y_ref, st_ref, x_ref, gam_ref, bet_ref, o_ref):
    st = jnp.sum(st_ref[...], axis=1)
    mean = jnp.sum(st[:, 0, :], axis=0) / TN
    var = jnp.sum(st[:, 1, :], axis=0) / TN - mean * mean
    yn = (y_ref[...] - mean[None, :]) * lax.rsqrt(var + EPS)[None, :] \
        * gam_ref[...][None, :] + bet_ref[...][None, :]
    o_ref[...] = jnp.maximum(yn, 0.0) + x_ref[...]


_bn_t = pl.pallas_call(
    _bn_t_body,
    grid=(T,),
    in_specs=[
        pl.BlockSpec((N, C), lambda t: (t, 0)),
        pl.BlockSpec((T, NB, 8, C), lambda t: (0, 0, 0, 0)),
        pl.BlockSpec((N, C), lambda t: (t, 0)),
        pl.BlockSpec((C,), lambda t: (0,)),
        pl.BlockSpec((C,), lambda t: (0,)),
    ],
    out_specs=pl.BlockSpec((N, C), lambda t: (t, 0)),
    out_shape=jax.ShapeDtypeStruct((TN, C), jnp.float32),
)


def kernel(x, edge_index, gcn_W, gcn_b, bn_s_gamma, bn_s_beta,
           conv_W, conv_b, bn_t_gamma, bn_t_beta):
    xf = x.reshape(TN, C)
    src = edge_index[0].reshape(NS, EPT)
    dst = edge_index[1].reshape(NS, EPT)
    pad_s = jnp.zeros((NS, EPT_PAD - EPT), jnp.int32)
    pad_d = jnp.full((NS, EPT_PAD - EPT), DUMMY, jnp.int32)
    srcp = jnp.concatenate([src, pad_s], axis=1).reshape(NS, NCHUNK, CHUNK)
    dstp = jnp.concatenate([dst, pad_d], axis=1).reshape(NS, NCHUNK, CHUNK)
    pkp = (srcp | (dstp << 14)).reshape(NS, NCHUNK * CHUNK)
    z128 = jnp.zeros((NPAD, C), jnp.float32)
    dst32 = jnp.concatenate(
        [dst.reshape(NC * NS, EPTD),
         jnp.full((NC * NS, EPTD_PAD - EPTD), DUMMY, jnp.int32)], axis=1
    ).reshape(NC, NS, NCHUNKD, CHUNK)
    ones128 = jnp.ones((CHUNK, C), jnp.float32)

    sck = _sc_kernels()
    deg = sck["deg"](dst32, ones128, z128)
    hp = _lin(xf, gcn_W, deg)
    s = sck["scatter"](hp, pkp, z128)
    hseq = _bn_s(s, hp, deg, gcn_b, bn_s_gamma, bn_s_beta)
    wk = jnp.transpose(conv_W, (2, 1, 0))
    y, st = _tconv(hseq, hseq, hseq, wk, conv_b)
    out = _bn_t(y, st, xf, bn_t_gamma, bn_t_beta)
    return out.reshape(1, T, N, C)


# speedup vs baseline: 1.0130x; 1.0130x over previous
import jax
import jax.numpy as jnp
from jax import lax
from jax.experimental import pallas as pl
from jax.experimental.pallas import tpu as pltpu
from jax.experimental.pallas import tpu_sc as plsc

N = 10000
E = 160000
T = 12
C = 128
TN = T * N
EPS = 1e-5

NC, NS = 2, 16
CHUNK = 64
EPT = E // NS
NCHUNK = (EPT + CHUNK - 1) // CHUNK
EPT_PAD = NCHUNK * CHUNK
NPAD = 10112
RPT = NPAD // NS
DUMMY = N + 8
TPC = T // NC

EPTD = E // (NC * NS)
NCHUNKD = (EPTD + CHUNK - 1) // CHUNK
EPTD_PAD = NCHUNKD * CHUNK


def _deg_body(dstp, ones_in, z16, deg_out, dst_buf, ones_buf, deg_acc):
    cid = lax.axis_index("c")
    sid = lax.axis_index("s")
    pltpu.sync_copy(dstp.at[cid, sid], dst_buf)
    pltpu.sync_copy(ones_in, ones_buf)
    r0 = sid * RPT
    pltpu.sync_copy(z16.at[pl.ds(r0, RPT)], deg_acc.at[pl.ds(r0, RPT)])
    plsc.subcore_barrier()

    def chunk(j, carry):
        pltpu.sync_copy(ones_buf, deg_acc.at[dst_buf.at[j]], add=True)
        return carry
    lax.fori_loop(0, NCHUNKD, chunk, 0)
    plsc.subcore_barrier()
    pltpu.sync_copy(deg_acc.at[pl.ds(r0, RPT)],
                    deg_out.at[cid, pl.ds(r0, RPT)])


_SC_KERNELS = {}


def _sc_kernels():
    if "deg" not in _SC_KERNELS:
        mesh = plsc.VectorSubcoreMesh(
            core_axis_name="c", subcore_axis_name="s",
            num_cores=NC, num_subcores=NS)
        _SC_KERNELS["deg"] = pl.kernel(
            _deg_body,
            out_type=jax.ShapeDtypeStruct((NC, NPAD, C), jnp.float32),
            mesh=mesh,
            scratch_types=[
                pltpu.VMEM((NCHUNKD, CHUNK), jnp.int32),
                pltpu.VMEM((CHUNK, C), jnp.float32),
                pltpu.VMEM_SHARED((NPAD, C), jnp.float32),
            ],
        )
        _SC_KERNELS["scatter"] = pl.kernel(
            _scatter_body,
            out_type=jax.ShapeDtypeStruct((T, NPAD, C), jnp.float32),
            mesh=mesh,
            scratch_types=[
                pltpu.VMEM((NCHUNK * CHUNK,), jnp.int32),
                pltpu.VMEM((4, CHUNK), jnp.int32),
                pltpu.VMEM((4, CHUNK), jnp.int32),
                pltpu.VMEM((4, CHUNK, C), jnp.float32),
                pltpu.VMEM_SHARED((NPAD, C), jnp.float32),
            ] + [pltpu.SemaphoreType.DMA] * 8,
        )
    return _SC_KERNELS


def _scatter_body(hp, pkp, zbf, s_out, pk_buf, sidx, didx, gbuf, acc,
                  g0, g1, g2, g3, s0, s1, s2, s3):
    cid = lax.axis_index("c")
    sid = lax.axis_index("s")
    r0 = sid * RPT
    pltpu.sync_copy(pkp.at[sid], pk_buf)
    mask = jnp.int32((1 << 14) - 1)
    gsems = (g0, g1, g2, g3)
    ssems = (s0, s1, s2, s3)

    def fire_gather(c, slot, toff):
        base = c * CHUNK
        for kk in range(CHUNK // 16):
            sl = pl.ds(kk * 16, 16)
            v = pk_buf[pl.ds(base + kk * 16, 16)]
            sidx[slot, sl] = (v & mask) + toff
            didx[slot, sl] = lax.shift_right_logical(v, 14)
        pltpu.async_copy(hp.at[sidx.at[slot]], gbuf.at[slot], gsems[slot])

    def wait_gather(slot):
        pltpu.make_async_copy(hp.at[sidx.at[slot]], gbuf.at[slot],
                              gsems[slot]).wait()

    def fire_scatter(slot):
        pltpu.async_copy(gbuf.at[slot], acc.at[didx.at[slot]], ssems[slot],
                         add=True)

    def wait_scatter(slot):
        pltpu.make_async_copy(gbuf.at[slot], acc.at[didx.at[slot]],
                              ssems[slot]).wait()

    for tl in range(TPC):
        t = cid * TPC + tl
        toff = t * jnp.int32(N)
        pltpu.sync_copy(zbf.at[pl.ds(r0, RPT)], acc.at[pl.ds(r0, RPT)])
        plsc.subcore_barrier()

        fire_gather(0, 0, toff)
        fire_gather(1, 1, toff)
        fire_gather(2, 2, toff)

        def quad(i, carry):
            c = 4 * i
            for off in range(4):
                cc = c + off
                slot = off
                nslot = (off + 3) % 4

                @pl.when(cc < NCHUNK)
                def _():
                    wait_gather(slot)
                    fire_scatter(slot)

                @pl.when((cc + 3 < NCHUNK) & (cc >= 1))
                def _():
                    wait_scatter(nslot)
                    fire_gather(cc + 3, nslot, toff)

                @pl.when((cc + 3 < NCHUNK) & (cc < 1))
                def _():
                    fire_gather(cc + 3, nslot, toff)
            return carry
        lax.fori_loop(0, (NCHUNK + 3) // 4, quad, 0)
        for slot in range(4):
            wait_scatter(slot)
        plsc.subcore_barrier()
        pltpu.sync_copy(acc.at[pl.ds(r0, RPT)], s_out.at[t, pl.ds(r0, RPT)])


RB = 2000


def _lin_body(x_ref, w_ref, deg_ref, o_ref):
    dinv = lax.rsqrt(deg_ref[0, :, 0:1] + deg_ref[1, :, 0:1] + 1.0)
    h = lax.dot_general(x_ref[...], w_ref[...], (((1,), (0,)), ((), ())),
                        precision=lax.Precision.DEFAULT,
                        preferred_element_type=jnp.float32)
    o_ref[...] = h * dinv


_lin = pl.pallas_call(
    _lin_body,
    grid=(N // RB, T),
    in_specs=[
        pl.BlockSpec((RB, C), lambda j, t: (t * (N // RB) + j, 0)),
        pl.BlockSpec((C, C), lambda j, t: (0, 0)),
        pl.BlockSpec((2, RB, C), lambda j, t: (0, j, 0)),
    ],
    out_specs=pl.BlockSpec((RB, C), lambda j, t: (t * (N // RB) + j, 0)),
    out_shape=jax.ShapeDtypeStruct((TN, C), jnp.float32),
)


def _bn_s_body(s_ref, hp_ref, deg_ref, b_ref, gam_ref, bet_ref, o_ref):
    dinv = lax.rsqrt(deg_ref[0, 0:N, 0:1] + deg_ref[1, 0:N, 0:1] + 1.0)
    sf = s_ref[0, 0:N, :].astype(jnp.float32)
    hf = hp_ref[...].astype(jnp.float32)
    g = dinv * (sf + hf) + b_ref[...][None, :]
    mean = jnp.mean(g, axis=0, keepdims=True)
    var = jnp.mean(jnp.square(g - mean), axis=0, keepdims=True)
    hs = (g - mean) * lax.rsqrt(var + EPS) * gam_ref[...][None, :] \
        + bet_ref[...][None, :]
    o_ref[...] = jnp.maximum(hs, 0.0)


_bn_s = pl.pallas_call(
    _bn_s_body,
    grid=(T,),
    in_specs=[
        pl.BlockSpec((1, NPAD, C), lambda t: (t, 0, 0)),
        pl.BlockSpec((N, C), lambda t: (t, 0)),
        pl.BlockSpec((2, NPAD, C), lambda t: (0, 0, 0)),
        pl.BlockSpec((C,), lambda t: (0,)),
        pl.BlockSpec((C,), lambda t: (0,)),
        pl.BlockSpec((C,), lambda t: (0,)),
    ],
    out_specs=pl.BlockSpec((N, C), lambda t: (t, 0)),
    out_shape=jax.ShapeDtypeStruct((TN, C), jnp.float32),
)


NB = 5
RBT = N // NB


def _tconv_body(h_ref, wk_ref, cb_ref, y_ref, st_ref, win_ref):
    t = pl.program_id(0)
    b = pl.program_id(1)
    rows = pl.ds(pl.multiple_of(b * RBT, 8), RBT)
    cur = h_ref[...]

    def dot(a, w):
        return lax.dot_general(a, w, (((1,), (0,)), ((), ())),
                               precision=lax.Precision.DEFAULT,
                               preferred_element_type=jnp.float32)

    def emit(w2_ref, w1_ref, wcur_ref):
        @pl.when(t >= 1)
        def _():
            w2 = w2_ref[rows, :]
            w1 = w1_ref[rows, :]
            zmat = jnp.zeros((RBT, C), jnp.float32)
            acc = dot(w1, wk_ref[1]) + cb_ref[...][None, :]
            acc = acc + jnp.where(t >= 2, dot(w2, wk_ref[0]), zmat)
            acc = acc + jnp.where(t <= T - 1, dot(cur, wk_ref[2]), zmat)
            y_ref[...] = acc
            s1 = jnp.sum(acc, axis=0)
            s2 = jnp.sum(acc * acc, axis=0)
            st_ref[0, 0] = jnp.concatenate(
                [s1[None, :], s2[None, :], jnp.zeros((6, C), jnp.float32)],
                axis=0)

        @pl.when(t <= T - 1)
        def _():
            wcur_ref[rows, :] = cur

    wA = win_ref.at[0]
    wB = win_ref.at[1]

    @pl.when(lax.rem(t, 2) == 0)
    def _():
        emit(wA, wB, wA)

    @pl.when(lax.rem(t, 2) == 1)
    def _():
        emit(wB, wA, wB)


_tconv = pl.pallas_call(
    _tconv_body,
    grid=(T + 1, NB),
    in_specs=[
        pl.BlockSpec((RBT, C), lambda t, b: (jnp.minimum(t, T - 1) * NB + b, 0)),
        pl.BlockSpec((3, C, C), lambda t, b: (0, 0, 0)),
        pl.BlockSpec((C,), lambda t, b: (0,)),
    ],
    out_specs=[
        pl.BlockSpec((RBT, C), lambda t, b: (jnp.maximum(t - 1, 0) * NB + b, 0)),
        pl.BlockSpec((1, 1, 8, C), lambda t, b: (jnp.maximum(t - 1, 0), b, 0, 0)),
    ],
    out_shape=[
        jax.ShapeDtypeStruct((TN, C), jnp.float32),
        jax.ShapeDtypeStruct((T, NB, 8, C), jnp.float32),
    ],
    scratch_shapes=[pltpu.VMEM((2, N, C), jnp.float32)],
)


def _bn_t_body(y_ref, st_ref, x_ref, gam_ref, bet_ref, o_ref):
    st = jnp.sum(st_ref[...], axis=1)
    mean = jnp.sum(st[:, 0, :], axis=0) / TN
    var = jnp.sum(st[:, 1, :], axis=0) / TN - mean * mean
    yn = (y_ref[...] - mean[None, :]) * lax.rsqrt(var + EPS)[None, :] \
        * gam_ref[...][None, :] + bet_ref[...][None, :]
    o_ref[...] = jnp.maximum(yn, 0.0) + x_ref[...]


_bn_t = pl.pallas_call(
    _bn_t_body,
    grid=(T,),
    in_specs=[
        pl.BlockSpec((N, C), lambda t: (t, 0)),
        pl.BlockSpec((T, NB, 8, C), lambda t: (0, 0, 0, 0)),
        pl.BlockSpec((N, C), lambda t: (t, 0)),
        pl.BlockSpec((C,), lambda t: (0,)),
        pl.BlockSpec((C,), lambda t: (0,)),
    ],
    out_specs=pl.BlockSpec((N, C), lambda t: (t, 0)),
    out_shape=jax.ShapeDtypeStruct((TN, C), jnp.float32),
)


def kernel(x, edge_index, gcn_W, gcn_b, bn_s_gamma, bn_s_beta,
           conv_W, conv_b, bn_t_gamma, bn_t_beta):
    xf = x.reshape(TN, C)
    src = edge_index[0].reshape(NS, EPT)
    dst = edge_index[1].reshape(NS, EPT)
    pad_s = jnp.zeros((NS, EPT_PAD - EPT), jnp.int32)
    pad_d = jnp.full((NS, EPT_PAD - EPT), DUMMY, jnp.int32)
    srcp = jnp.concatenate([src, pad_s], axis=1).reshape(NS, NCHUNK, CHUNK)
    dstp = jnp.concatenate([dst, pad_d], axis=1).reshape(NS, NCHUNK, CHUNK)
    pkp = (srcp | (dstp << 14)).reshape(NS, NCHUNK * CHUNK)
    z128 = jnp.zeros((NPAD, C), jnp.float32)
    dst32 = jnp.concatenate(
        [dst.reshape(NC * NS, EPTD),
         jnp.full((NC * NS, EPTD_PAD - EPTD), DUMMY, jnp.int32)], axis=1
    ).reshape(NC, NS, NCHUNKD, CHUNK)
    ones128 = jnp.ones((CHUNK, C), jnp.float32)

    sck = _sc_kernels()
    deg = sck["deg"](dst32, ones128, z128)
    hp = _lin(xf, gcn_W, deg)
    s = sck["scatter"](hp, pkp, z128)
    hseq = _bn_s(s, hp, deg, gcn_b, bn_s_gamma, bn_s_beta)
    wk = jnp.transpose(conv_W, (2, 1, 0))
    y, st = _tconv(hseq, wk, conv_b)
    out = _bn_t(y, st, xf, bn_t_gamma, bn_t_beta)
    return out.reshape(1, T, N, C)


# speedup vs baseline: 1.0233x; 1.0101x over previous
import jax
import jax.numpy as jnp
from jax import lax
from jax.experimental import pallas as pl
from jax.experimental.pallas import tpu as pltpu
from jax.experimental.pallas import tpu_sc as plsc

N = 10000
E = 160000
T = 12
C = 128
TN = T * N
EPS = 1e-5

NC, NS = 2, 16
CHUNK = 64
EPT = E // NS
NCHUNK = (EPT + CHUNK - 1) // CHUNK
EPT_PAD = NCHUNK * CHUNK
NPAD = 10112
RPT = NPAD // NS
DUMMY = N + 8
TPC = T // NC

EPTD = E // (NC * NS)
NCHUNKD = (EPTD + CHUNK - 1) // CHUNK
EPTD_PAD = NCHUNKD * CHUNK


def _deg_body(dstp, ones_in, z16, deg_out, dst_buf, ones_buf, deg_acc):
    cid = lax.axis_index("c")
    sid = lax.axis_index("s")
    pltpu.sync_copy(dstp.at[cid, sid], dst_buf)
    pltpu.sync_copy(ones_in, ones_buf)
    r0 = sid * RPT
    pltpu.sync_copy(z16.at[pl.ds(r0, RPT)], deg_acc.at[pl.ds(r0, RPT)])
    plsc.subcore_barrier()

    def chunk(j, carry):
        pltpu.sync_copy(ones_buf, deg_acc.at[dst_buf.at[j]], add=True)
        return carry
    lax.fori_loop(0, NCHUNKD, chunk, 0)
    plsc.subcore_barrier()
    pltpu.sync_copy(deg_acc.at[pl.ds(r0, RPT)],
                    deg_out.at[cid, pl.ds(r0, RPT)])


_SC_KERNELS = {}


def _sc_kernels():
    if "deg" not in _SC_KERNELS:
        mesh = plsc.VectorSubcoreMesh(
            core_axis_name="c", subcore_axis_name="s",
            num_cores=NC, num_subcores=NS)
        _SC_KERNELS["deg"] = pl.kernel(
            _deg_body,
            out_type=jax.ShapeDtypeStruct((NC, NPAD, C), jnp.float32),
            mesh=mesh,
            scratch_types=[
                pltpu.VMEM((NCHUNKD, CHUNK), jnp.int32),
                pltpu.VMEM((CHUNK, C), jnp.float32),
                pltpu.VMEM_SHARED((NPAD, C), jnp.float32),
            ],
        )
        _SC_KERNELS["scatter"] = pl.kernel(
            _scatter_body,
            out_type=jax.ShapeDtypeStruct((T, NPAD, C), jnp.float32),
            mesh=mesh,
            scratch_types=[
                pltpu.VMEM((NCHUNK * CHUNK,), jnp.int32),
                pltpu.VMEM((4, CHUNK), jnp.int32),
                pltpu.VMEM((4, CHUNK), jnp.int32),
                pltpu.VMEM((4, CHUNK, C), jnp.float32),
                pltpu.VMEM_SHARED((NPAD, C), jnp.float32),
            ] + [pltpu.SemaphoreType.DMA] * 8,
        )
    return _SC_KERNELS


def _scatter_body(hp, pkp, zbf, s_out, pk_buf, sidx, didx, gbuf, acc,
                  g0, g1, g2, g3, s0, s1, s2, s3):
    cid = lax.axis_index("c")
    sid = lax.axis_index("s")
    r0 = sid * RPT
    pltpu.sync_copy(pkp.at[sid], pk_buf)
    mask = jnp.int32((1 << 14) - 1)
    gsems = (g0, g1, g2, g3)
    ssems = (s0, s1, s2, s3)

    def fire_gather(c, slot, toff):
        base = c * CHUNK
        for kk in range(CHUNK // 16):
            sl = pl.ds(kk * 16, 16)
            v = pk_buf[pl.ds(base + kk * 16, 16)]
            sidx[slot, sl] = (v & mask) + toff
            didx[slot, sl] = lax.shift_right_logical(v, 14)
        pltpu.async_copy(hp.at[sidx.at[slot]], gbuf.at[slot], gsems[slot])

    def wait_gather(slot):
        pltpu.make_async_copy(hp.at[sidx.at[slot]], gbuf.at[slot],
                              gsems[slot]).wait()

    def fire_scatter(slot):
        pltpu.async_copy(gbuf.at[slot], acc.at[didx.at[slot]], ssems[slot],
                         add=True)

    def wait_scatter(slot):
        pltpu.make_async_copy(gbuf.at[slot], acc.at[didx.at[slot]],
                              ssems[slot]).wait()

    for tl in range(TPC):
        t = cid * TPC + tl
        toff = t * jnp.int32(N)
        pltpu.sync_copy(zbf.at[pl.ds(r0, RPT)], acc.at[pl.ds(r0, RPT)])
        plsc.subcore_barrier()

        fire_gather(0, 0, toff)
        fire_gather(1, 1, toff)
        fire_gather(2, 2, toff)

        def quad(i, carry):
            c = 4 * i
            for off in range(4):
                cc = c + off
                slot = off
                nslot = (off + 3) % 4

                @pl.when(cc < NCHUNK)
                def _():
                    wait_gather(slot)
                    fire_scatter(slot)

                @pl.when((cc + 3 < NCHUNK) & (cc >= 1))
                def _():
                    wait_scatter(nslot)
                    fire_gather(cc + 3, nslot, toff)

                @pl.when((cc + 3 < NCHUNK) & (cc < 1))
                def _():
                    fire_gather(cc + 3, nslot, toff)
            return carry
        lax.fori_loop(0, (NCHUNK + 3) // 4, quad, 0)
        for slot in range(4):
            wait_scatter(slot)
        plsc.subcore_barrier()
        pltpu.sync_copy(acc.at[pl.ds(r0, RPT)], s_out.at[t, pl.ds(r0, RPT)])


RB = 2000


def _lin_body(x_ref, w_ref, deg_ref, o_ref):
    dinv = lax.rsqrt(deg_ref[0, :, 0:1] + deg_ref[1, :, 0:1] + 1.0)
    h = lax.dot_general(x_ref[...], w_ref[...], (((1,), (0,)), ((), ())),
                        precision=lax.Precision.DEFAULT,
                        preferred_element_type=jnp.float32)
    o_ref[...] = h * dinv


_lin = pl.pallas_call(
    _lin_body,
    grid=(N // RB, T),
    in_specs=[
        pl.BlockSpec((RB, C), lambda j, t: (t * (N // RB) + j, 0)),
        pl.BlockSpec((C, C), lambda j, t: (0, 0)),
        pl.BlockSpec((2, RB, C), lambda j, t: (0, j, 0)),
    ],
    out_specs=pl.BlockSpec((RB, C), lambda j, t: (t * (N // RB) + j, 0)),
    out_shape=jax.ShapeDtypeStruct((TN, C), jnp.float32),
)


def _bn_s_body(s_ref, hp_ref, deg_ref, b_ref, gam_ref, bet_ref, o_ref):
    dinv = lax.rsqrt(deg_ref[0, 0:N, 0:1] + deg_ref[1, 0:N, 0:1] + 1.0)
    sf = s_ref[0, 0:N, :].astype(jnp.float32)
    hf = hp_ref[...].astype(jnp.float32)
    g = dinv * (sf + hf) + b_ref[...][None, :]
    mean = jnp.mean(g, axis=0, keepdims=True)
    var = jnp.mean(jnp.square(g - mean), axis=0, keepdims=True)
    hs = (g - mean) * lax.rsqrt(var + EPS) * gam_ref[...][None, :] \
        + bet_ref[...][None, :]
    o_ref[...] = jnp.maximum(hs, 0.0).astype(jnp.bfloat16)


_bn_s = pl.pallas_call(
    _bn_s_body,
    grid=(T,),
    in_specs=[
        pl.BlockSpec((1, NPAD, C), lambda t: (t, 0, 0)),
        pl.BlockSpec((N, C), lambda t: (t, 0)),
        pl.BlockSpec((2, NPAD, C), lambda t: (0, 0, 0)),
        pl.BlockSpec((C,), lambda t: (0,)),
        pl.BlockSpec((C,), lambda t: (0,)),
        pl.BlockSpec((C,), lambda t: (0,)),
    ],
    out_specs=pl.BlockSpec((N, C), lambda t: (t, 0)),
    out_shape=jax.ShapeDtypeStruct((TN, C), jnp.bfloat16),
)


NB = 5
RBT = N // NB


def _tconv_body(h_ref, wk_ref, cb_ref, y_ref, st_ref, win_ref):
    t = pl.program_id(0)
    b = pl.program_id(1)
    rows = pl.ds(pl.multiple_of(b * RBT, 8), RBT)
    cur = h_ref[...]

    def dot(a, w):
        return lax.dot_general(a, w, (((1,), (0,)), ((), ())),
                               precision=lax.Precision.DEFAULT,
                               preferred_element_type=jnp.float32)

    def emit(w2_ref, w1_ref, wcur_ref):
        @pl.when(t >= 1)
        def _():
            w2 = w2_ref[rows, :]
            w1 = w1_ref[rows, :]
            zmat = jnp.zeros((RBT, C), jnp.float32)
            acc = dot(w1, wk_ref[1]) + cb_ref[...][None, :]
            acc = acc + jnp.where(t >= 2, dot(w2, wk_ref[0]), zmat)
            acc = acc + jnp.where(t <= T - 1, dot(cur, wk_ref[2]), zmat)
            y_ref[...] = acc.astype(jnp.bfloat16)
            s1 = jnp.sum(acc, axis=0)
            s2 = jnp.sum(acc * acc, axis=0)
            st_ref[0, 0] = jnp.concatenate(
                [s1[None, :], s2[None, :], jnp.zeros((6, C), jnp.float32)],
                axis=0)

        @pl.when(t <= T - 1)
        def _():
            wcur_ref[rows, :] = cur

    wA = win_ref.at[0]
    wB = win_ref.at[1]

    @pl.when(lax.rem(t, 2) == 0)
    def _():
        emit(wA, wB, wA)

    @pl.when(lax.rem(t, 2) == 1)
    def _():
        emit(wB, wA, wB)


_tconv = pl.pallas_call(
    _tconv_body,
    grid=(T + 1, NB),
    in_specs=[
        pl.BlockSpec((RBT, C), lambda t, b: (jnp.minimum(t, T - 1) * NB + b, 0)),
        pl.BlockSpec((3, C, C), lambda t, b: (0, 0, 0)),
        pl.BlockSpec((C,), lambda t, b: (0,)),
    ],
    out_specs=[
        pl.BlockSpec((RBT, C), lambda t, b: (jnp.maximum(t - 1, 0) * NB + b, 0)),
        pl.BlockSpec((1, 1, 8, C), lambda t, b: (jnp.maximum(t - 1, 0), b, 0, 0)),
    ],
    out_shape=[
        jax.ShapeDtypeStruct((TN, C), jnp.bfloat16),
        jax.ShapeDtypeStruct((T, NB, 8, C), jnp.float32),
    ],
    scratch_shapes=[pltpu.VMEM((2, N, C), jnp.bfloat16)],
)


def _bn_t_body(y_ref, st_ref, x_ref, gam_ref, bet_ref, o_ref):
    st = jnp.sum(st_ref[...], axis=1)
    mean = jnp.sum(st[:, 0, :], axis=0) / TN
    var = jnp.sum(st[:, 1, :], axis=0) / TN - mean * mean
    yn = (y_ref[...].astype(jnp.float32) - mean[None, :]) * lax.rsqrt(var + EPS)[None, :] \
        * gam_ref[...][None, :] + bet_ref[...][None, :]
    o_ref[...] = jnp.maximum(yn, 0.0) + x_ref[...]


_bn_t = pl.pallas_call(
    _bn_t_body,
    grid=(T,),
    in_specs=[
        pl.BlockSpec((N, C), lambda t: (t, 0)),
        pl.BlockSpec((T, NB, 8, C), lambda t: (0, 0, 0, 0)),
        pl.BlockSpec((N, C), lambda t: (t, 0)),
        pl.BlockSpec((C,), lambda t: (0,)),
        pl.BlockSpec((C,), lambda t: (0,)),
    ],
    out_specs=pl.BlockSpec((N, C), lambda t: (t, 0)),
    out_shape=jax.ShapeDtypeStruct((TN, C), jnp.float32),
)


def kernel(x, edge_index, gcn_W, gcn_b, bn_s_gamma, bn_s_beta,
           conv_W, conv_b, bn_t_gamma, bn_t_beta):
    xf = x.reshape(TN, C)
    src = edge_index[0].reshape(NS, EPT)
    dst = edge_index[1].reshape(NS, EPT)
    pad_s = jnp.zeros((NS, EPT_PAD - EPT), jnp.int32)
    pad_d = jnp.full((NS, EPT_PAD - EPT), DUMMY, jnp.int32)
    srcp = jnp.concatenate([src, pad_s], axis=1).reshape(NS, NCHUNK, CHUNK)
    dstp = jnp.concatenate([dst, pad_d], axis=1).reshape(NS, NCHUNK, CHUNK)
    pkp = (srcp | (dstp << 14)).reshape(NS, NCHUNK * CHUNK)
    z128 = jnp.zeros((NPAD, C), jnp.float32)
    dst32 = jnp.concatenate(
        [dst.reshape(NC * NS, EPTD),
         jnp.full((NC * NS, EPTD_PAD - EPTD), DUMMY, jnp.int32)], axis=1
    ).reshape(NC, NS, NCHUNKD, CHUNK)
    ones128 = jnp.ones((CHUNK, C), jnp.float32)

    sck = _sc_kernels()
    deg = sck["deg"](dst32, ones128, z128)
    hp = _lin(xf, gcn_W, deg)
    s = sck["scatter"](hp, pkp, z128)
    hseq = _bn_s(s, hp, deg, gcn_b, bn_s_gamma, bn_s_beta)
    wk = jnp.transpose(conv_W, (2, 1, 0)).astype(jnp.bfloat16)
    y, st = _tconv(hseq, wk, conv_b)
    out = _bn_t(y, st, xf, bn_t_gamma, bn_t_beta)
    return out.reshape(1, T, N, C)
